# Initial kernel scaffold; baseline (speedup 1.0000x reference)
import jax
import jax.numpy as jnp
from jax.experimental import pallas as pl


def kernel(x, conv1_w, conv1_b, bn1_g, bn1_b, conv2_w, conv2_b, bn2_g, bn2_b, gcn1_W, gcn1_b, gcn2_W, gcn2_b, fc1_W, fc1_b, fc2_W, fc2_b):
    raise NotImplementedError("write your pallas kernel here")


# speedup vs baseline: 32.9862x; 32.9862x over previous
import numpy as np
import jax
import jax.numpy as jnp
from jax.experimental import pallas as pl

B, C_IN, T = 64, 16, 2048
HIDDEN, OUT = 128, 32
WINDOW = 8
TR = T // 4
EPS = 1e-5


def _band_matrix():
    t = np.arange(TR)
    deg = np.minimum(t, WINDOW) + np.minimum(TR - 1 - t, WINDOW) + 1.0
    dis = 1.0 / np.sqrt(deg)
    band = (np.abs(t[:, None] - t[None, :]) <= WINDOW).astype(np.float32)
    return (dis[:, None] * dis[None, :] * band).astype(np.float32)


def _conv1_packed(conv1_w, conv1_b, bn1_g, bn1_b):
    s = bn1_g / jnp.sqrt(1.0 + EPS)
    w = conv1_w * s[:, None, None]
    W = jnp.zeros((64, 128), jnp.float32)
    for j in range(4):
        for k in range(5):
            p = (j + k - 2) % 4
            sh = (j + k - 2) // 4
            if sh == 0:
                m = p
            elif sh == -1:
                m = {2: 4, 3: 5}[p]
            else:
                m = {0: 6, 1: 7}[p]
            W = W.at[16 * j:16 * (j + 1), 16 * m:16 * (m + 1)].add(w[:, :, k])
    b = s * conv1_b + bn1_b
    bias = jnp.tile(b, 4).reshape(64, 1)
    return W, bias


def _conv2_packed(conv2_w, conv2_b, bn2_g, bn2_b):
    s = bn2_g / jnp.sqrt(1.0 + EPS)
    w = conv2_w * s[:, None, None]
    W = jnp.zeros((64, 96), jnp.float32)
    even_slots = [2, 3, 0, 1, 4]
    odd_slots = [3, 0, 1, 4, 5]
    for k in range(5):
        W = W.at[0:32, 16 * even_slots[k]:16 * (even_slots[k] + 1)].add(w[:, :, k])
        W = W.at[32:64, 16 * odd_slots[k]:16 * (odd_slots[k] + 1)].add(w[:, :, k])
    b = s * conv2_b + bn2_b
    bias = jnp.concatenate([b, b]).reshape(64, 1)
    return W, bias


def _shift_m1(x):
    return jnp.concatenate([jnp.zeros((x.shape[0], 1), x.dtype), x[:, :-1]], axis=1)


def _shift_p1(x):
    return jnp.concatenate([x[:, 1:], jnp.zeros((x.shape[0], 1), x.dtype)], axis=1)


def _fused_kernel(xph_ref, w1_ref, b1_ref, w2_ref, b2_ref, band_ref,
                  g1w_ref, g1b_ref, g2w_ref, g2b_ref,
                  f1w_ref, f1b_ref, f2w_ref, f2b_ref, out_ref):
    xph = xph_ref[0]
    x0 = xph[0:16]
    x1 = xph[16:32]
    x2 = xph[32:48]
    x3 = xph[48:64]
    xcat = jnp.concatenate([
        x0, x1, x2, x3,
        _shift_m1(x2), _shift_m1(x3),
        _shift_p1(x0), _shift_p1(x1),
    ], axis=0)

    y = jnp.maximum(
        jnp.dot(w1_ref[:], xcat, preferred_element_type=jnp.float32) + b1_ref[:],
        0.0)
    qe = jnp.maximum(y[0:16], y[16:32])
    qo = jnp.maximum(y[32:48], y[48:64])

    qcat = jnp.concatenate([
        qe, qo, _shift_m1(qe), _shift_m1(qo), _shift_p1(qe), _shift_p1(qo),
    ], axis=0)
    z = jnp.maximum(
        jnp.dot(w2_ref[:], qcat, preferred_element_type=jnp.float32) + b2_ref[:],
        0.0)
    xg = jnp.maximum(z[0:32], z[32:64])

    A = band_ref[:]

    xa = jnp.dot(xg, A, preferred_element_type=jnp.float32)
    h1 = jnp.dot(g1w_ref[:], xa, preferred_element_type=jnp.float32)
    h1 = jnp.maximum(h1 + g1b_ref[:], 0.0)

    ha = jnp.dot(h1, A, preferred_element_type=jnp.float32)
    h2 = jnp.dot(g2w_ref[:], ha, preferred_element_type=jnp.float32)
    h2 = jnp.maximum(h2 + g2b_ref[:], 0.0)

    pooled = jnp.mean(h2, axis=1, keepdims=True)
    hfc = jnp.maximum(
        jnp.dot(f1w_ref[:], pooled, preferred_element_type=jnp.float32) + f1b_ref[:],
        0.0)
    logits = jnp.dot(f2w_ref[:], hfc, preferred_element_type=jnp.float32) + f2b_ref[:]
    out_ref[0] = logits


def kernel(x, conv1_w, conv1_b, bn1_g, bn1_b, conv2_w, conv2_b, bn2_g, bn2_b,
           gcn1_W, gcn1_b, gcn2_W, gcn2_b, fc1_W, fc1_b, fc2_W, fc2_b):
    xph = x.reshape(B, C_IN, TR, 4).transpose(0, 3, 1, 2).reshape(B, 64, TR)

    w1, b1 = _conv1_packed(conv1_w, conv1_b, bn1_g, bn1_b)
    w2, b2 = _conv2_packed(conv2_w, conv2_b, bn2_g, bn2_b)
    band = jnp.asarray(_band_matrix())

    g1w = gcn1_W.T
    g1b = gcn1_b.reshape(HIDDEN, 1)
    g2w = gcn2_W.T
    g2b = gcn2_b.reshape(HIDDEN, 1)
    f1w = fc1_W.T
    f1b = fc1_b.reshape(HIDDEN, 1)
    f2w = fc2_W.T
    f2b = fc2_b.reshape(OUT, 1)

    const = lambda shape: pl.BlockSpec(shape, lambda b: (0,) * len(shape))
    out = pl.pallas_call(
        _fused_kernel,
        grid=(B,),
        in_specs=[
            pl.BlockSpec((1, 64, TR), lambda b: (b, 0, 0)),
            const((64, 128)), const((64, 1)),
            const((64, 96)), const((64, 1)),
            const((TR, TR)),
            const((HIDDEN, 32)), const((HIDDEN, 1)),
            const((HIDDEN, HIDDEN)), const((HIDDEN, 1)),
            const((HIDDEN, HIDDEN)), const((HIDDEN, 1)),
            const((OUT, HIDDEN)), const((OUT, 1)),
        ],
        out_specs=pl.BlockSpec((1, OUT, 1), lambda b: (b, 0, 0)),
        out_shape=jax.ShapeDtypeStruct((B, OUT, 1), jnp.float32),
    )(xph, w1, b1, w2, b2, band, g1w, g1b, g2w, g2b, f1w, f1b, f2w, f2b)
    return out.reshape(B, OUT)


# speedup vs baseline: 52.3926x; 1.5883x over previous
import numpy as np
import jax
import jax.numpy as jnp
from jax.experimental import pallas as pl

B, C_IN, T = 64, 16, 2048
HIDDEN, OUT = 128, 32
WINDOW = 8
TR = T // 4
EPS = 1e-5
S = 8
G = B // S
L = S * TR


def _band_matrix():
    t = np.arange(TR)
    deg = np.minimum(t, WINDOW) + np.minimum(TR - 1 - t, WINDOW) + 1.0
    dis = 1.0 / np.sqrt(deg)
    band = (np.abs(t[:, None] - t[None, :]) <= WINDOW).astype(np.float32)
    return (dis[:, None] * dis[None, :] * band).astype(np.float32)


def _conv1_packed(conv1_w, conv1_b, bn1_g, bn1_b):
    s = bn1_g / jnp.sqrt(1.0 + EPS)
    w = conv1_w * s[:, None, None]
    W = jnp.zeros((64, 128), jnp.float32)
    for j in range(4):
        for k in range(5):
            p = (j + k - 2) % 4
            sh = (j + k - 2) // 4
            if sh == 0:
                m = p
            elif sh == -1:
                m = {2: 4, 3: 5}[p]
            else:
                m = {0: 6, 1: 7}[p]
            W = W.at[16 * j:16 * (j + 1), 16 * m:16 * (m + 1)].add(w[:, :, k])
    b = s * conv1_b + bn1_b
    bias = jnp.tile(b, 4).reshape(64, 1)
    return W, bias


def _conv2_packed(conv2_w, conv2_b, bn2_g, bn2_b):
    s = bn2_g / jnp.sqrt(1.0 + EPS)
    w = conv2_w * s[:, None, None]
    W = jnp.zeros((64, 96), jnp.float32)
    even_slots = [2, 3, 0, 1, 4]
    odd_slots = [3, 0, 1, 4, 5]
    for k in range(5):
        W = W.at[0:32, 16 * even_slots[k]:16 * (even_slots[k] + 1)].add(w[:, :, k])
        W = W.at[32:64, 16 * odd_slots[k]:16 * (odd_slots[k] + 1)].add(w[:, :, k])
    b = s * conv2_b + bn2_b
    bias = jnp.concatenate([b, b]).reshape(64, 1)
    return W, bias


def _fused_kernel(xph_ref, w1_ref, b1_ref, w2_ref, b2_ref, band_ref,
                  g1w_ref, g1b_ref, g2w_ref, g2b_ref,
                  f1w_ref, f1b_ref, f2w_ref, f2b_ref, out_ref):
    lane = jax.lax.broadcasted_iota(jnp.int32, (1, L), 1)
    ok_m1 = (lane % TR) != 0
    ok_p1 = (lane % TR) != TR - 1

    def shift_m1(x):
        sh = jnp.concatenate([jnp.zeros((x.shape[0], 1), x.dtype), x[:, :-1]], 1)
        return jnp.where(ok_m1, sh, 0.0)

    def shift_p1(x):
        sh = jnp.concatenate([x[:, 1:], jnp.zeros((x.shape[0], 1), x.dtype)], 1)
        return jnp.where(ok_p1, sh, 0.0)

    xph = xph_ref[0]
    x0 = xph[0:16]
    x1 = xph[16:32]
    x2 = xph[32:48]
    x3 = xph[48:64]
    xcat = jnp.concatenate([
        x0, x1, x2, x3,
        shift_m1(x2), shift_m1(x3),
        shift_p1(x0), shift_p1(x1),
    ], axis=0)

    y = jnp.maximum(
        jnp.dot(w1_ref[:], xcat, preferred_element_type=jnp.float32) + b1_ref[:],
        0.0)
    qe = jnp.maximum(y[0:16], y[16:32])
    qo = jnp.maximum(y[32:48], y[48:64])

    qcat = jnp.concatenate([
        qe, qo, shift_m1(qe), shift_m1(qo), shift_p1(qe), shift_p1(qo),
    ], axis=0)
    z = jnp.maximum(
        jnp.dot(w2_ref[:], qcat, preferred_element_type=jnp.float32) + b2_ref[:],
        0.0)
    xg = jnp.maximum(z[0:32], z[32:64])

    A = band_ref[:]

    def agg(h):
        return jnp.concatenate(
            [jnp.dot(h[:, i * TR:(i + 1) * TR], A,
                     preferred_element_type=jnp.float32) for i in range(S)],
            axis=1)

    xa = agg(xg)
    h1 = jnp.dot(g1w_ref[:], xa, preferred_element_type=jnp.float32)
    h1 = jnp.maximum(h1 + g1b_ref[:], 0.0)

    ha = agg(h1)
    h2 = jnp.dot(g2w_ref[:], ha, preferred_element_type=jnp.float32)
    h2 = jnp.maximum(h2 + g2b_ref[:], 0.0)

    pooled = jnp.concatenate(
        [jnp.mean(h2[:, i * TR:(i + 1) * TR], axis=1, keepdims=True)
         for i in range(S)], axis=1)
    hfc = jnp.maximum(
        jnp.dot(f1w_ref[:], pooled, preferred_element_type=jnp.float32) + f1b_ref[:],
        0.0)
    logits = jnp.dot(f2w_ref[:], hfc, preferred_element_type=jnp.float32) + f2b_ref[:]
    out_ref[0] = logits


def kernel(x, conv1_w, conv1_b, bn1_g, bn1_b, conv2_w, conv2_b, bn2_g, bn2_b,
           gcn1_W, gcn1_b, gcn2_W, gcn2_b, fc1_W, fc1_b, fc2_W, fc2_b):
    xph = x.reshape(B, C_IN, TR, 4).transpose(0, 3, 1, 2).reshape(B, 64, TR)
    xg2 = xph.reshape(G, S, 64, TR).transpose(0, 2, 1, 3).reshape(G, 64, L)

    w1, b1 = _conv1_packed(conv1_w, conv1_b, bn1_g, bn1_b)
    w2, b2 = _conv2_packed(conv2_w, conv2_b, bn2_g, bn2_b)
    band = jnp.asarray(_band_matrix())

    g1w = gcn1_W.T
    g1b = gcn1_b.reshape(HIDDEN, 1)
    g2w = gcn2_W.T
    g2b = gcn2_b.reshape(HIDDEN, 1)
    f1w = fc1_W.T
    f1b = fc1_b.reshape(HIDDEN, 1)
    f2w = fc2_W.T
    f2b = fc2_b.reshape(OUT, 1)

    const = lambda shape: pl.BlockSpec(shape, lambda b: (0,) * len(shape))
    out = pl.pallas_call(
        _fused_kernel,
        grid=(G,),
        in_specs=[
            pl.BlockSpec((1, 64, L), lambda b: (b, 0, 0)),
            const((64, 128)), const((64, 1)),
            const((64, 96)), const((64, 1)),
            const((TR, TR)),
            const((HIDDEN, 32)), const((HIDDEN, 1)),
            const((HIDDEN, HIDDEN)), const((HIDDEN, 1)),
            const((HIDDEN, HIDDEN)), const((HIDDEN, 1)),
            const((OUT, HIDDEN)), const((OUT, 1)),
        ],
        out_specs=pl.BlockSpec((1, OUT, S), lambda b: (b, 0, 0)),
        out_shape=jax.ShapeDtypeStruct((G, OUT, S), jnp.float32),
    )(xg2, w1, b1, w2, b2, band, g1w, g1b, g2w, g2b, f1w, f1b, f2w, f2b)
    return out.transpose(0, 2, 1).reshape(B, OUT)
